# baseline (speedup 1.0000x reference)
import jax
import jax.numpy as jnp
from jax import lax
from jax.experimental import pallas as pl
from jax.experimental.pallas import tpu as pltpu
from jax.experimental.pallas import tpu_sc as plsc

N_USERS = 10000
N_ITEMS = 40000
N = N_USERS + N_ITEMS
E = 800000
D = 64

NC = 2
NS = 16
H = N // NC
TRASH = H
ACC_ROWS = H + 88
B = 128
NBT = 6272
NBT_ALLOC = NBT + 2 * NS
E_PAD = NBT_ALLOC * B
NB = NBT // NS
REC = 3 * B

Z_PER_TILE = ACC_ROWS // NS
CP_PER_TILE = 1560


def _bcast_lane(v16, e):
    idx = jnp.full((16, 1), e, jnp.int32)
    return lax.gather(
        v16, idx,
        dimension_numbers=lax.GatherDimensionNumbers(
            offset_dims=(), collapsed_slice_dims=(0,), start_index_map=(0,)),
        slice_sizes=(1,),
        mode=lax.GatherScatterMode.PROMISE_IN_BOUNDS)


def _layer_body(x, pk, y,
                pkb0, pkb1, pkb2, pkb3, rowsb0, rowsb1, lidxb0, lidxb1,
                zbuf, acc, stg0, stg1, gat0, gat1, sct0, sct1):
    c = lax.axis_index("c")
    s = lax.axis_index("s")
    base_dst = c * H

    pkb = (pkb0, pkb1, pkb2, pkb3)
    rowsb = (rowsb0, rowsb1)
    lidxb = (lidxb0, lidxb1)
    stg = (stg0, stg1)
    gat = (gat0, gat1)
    sct = (sct0, sct1)

    def zrow(r, _):
        for k in range(4):
            zbuf[r, pl.ds(k * 16, 16)] = jnp.zeros((16,), jnp.float32)
        return 0
    lax.fori_loop(0, 32, zrow, 0)
    z0 = s * Z_PER_TILE
    def zcopy(i, _):
        pltpu.sync_copy(zbuf, acc.at[pl.ds(z0 + i * 32, 32)])
        return 0
    lax.fori_loop(0, Z_PER_TILE // 32, zcopy, 0)
    plsc.subcore_barrier()

    def stage(j, q):
        return pltpu.async_copy(
            pk.at[pl.ds((s + j * NS) * REC, REC)], pkb[q], stg[q % 2])

    def gather(q, p):
        return pltpu.async_copy(x.at[pkb[q].at[pl.ds(0, B)]], rowsb[p],
                                gat[p])

    def scatter(p):
        return pltpu.async_copy(rowsb[p], acc.at[lidxb[p]], sct[p], add=True)

    def compute(q, p):
        for g in range(B // 16):
            gb = g * 16
            d16 = pkb[q][pl.ds(B + gb, 16)]
            inr = (d16 >= base_dst) & (d16 < base_dst + H)
            lidxb[p][pl.ds(gb, 16)] = jnp.where(inr, d16 - base_dst, TRASH)
            v16 = lax.bitcast_convert_type(
                pkb[q][pl.ds(2 * B + gb, 16)], jnp.float32)
            for e in range(16):
                sv = _bcast_lane(v16, e)
                r = gb + e
                for k in range(4):
                    rowsb[p][r, pl.ds(k * 16, 16)] = (
                        rowsb[p][r, pl.ds(k * 16, 16)] * sv)

    stage(0, 0).wait()
    stage(1, 1).wait()

    def subiter(j, qa, qb, qc, qd):
        dg0 = gather(qa, 0)
        dg1 = gather(qb, 1)
        ds2 = stage(j + 2, qc)
        ds3 = stage(j + 3, qd)
        dg0.wait()
        compute(qa, 0)
        dc0 = scatter(0)
        dg1.wait()
        compute(qb, 1)
        dc1 = scatter(1)
        dc0.wait()
        dc1.wait()
        ds2.wait()
        ds3.wait()

    def quad(i, _):
        subiter(4 * i, 0, 1, 2, 3)
        subiter(4 * i + 2, 2, 3, 0, 1)
        return 0
    lax.fori_loop(0, NB // 4, quad, 0)

    plsc.subcore_barrier()
    r0 = s * CP_PER_TILE
    pltpu.sync_copy(acc.at[pl.ds(r0, CP_PER_TILE)],
                    y.at[pl.ds(base_dst + r0, CP_PER_TILE)])
    @pl.when(s == 0)
    def _():
        pltpu.sync_copy(acc.at[pl.ds(NS * CP_PER_TILE, 40)],
                        y.at[pl.ds(base_dst + NS * CP_PER_TILE, 40)])


def _sc_layer(x, pk):
    mesh = plsc.VectorSubcoreMesh(
        core_axis_name="c", subcore_axis_name="s",
        num_cores=NC, num_subcores=NS)
    return pl.kernel(
        _layer_body,
        out_type=jax.ShapeDtypeStruct((N, D), jnp.float32),
        mesh=mesh,
        compiler_params=pltpu.CompilerParams(use_tc_tiling_on_sc=False),
        scratch_types=[
            pltpu.VMEM((REC,), jnp.int32),
            pltpu.VMEM((REC,), jnp.int32),
            pltpu.VMEM((REC,), jnp.int32),
            pltpu.VMEM((REC,), jnp.int32),
            pltpu.VMEM((B, D), jnp.float32),
            pltpu.VMEM((B, D), jnp.float32),
            pltpu.VMEM((B,), jnp.int32),
            pltpu.VMEM((B,), jnp.int32),
            pltpu.VMEM((32, D), jnp.float32),
            pltpu.VMEM_SHARED((ACC_ROWS, D), jnp.float32),
            pltpu.SemaphoreType.DMA,
            pltpu.SemaphoreType.DMA,
            pltpu.SemaphoreType.DMA,
            pltpu.SemaphoreType.DMA,
            pltpu.SemaphoreType.DMA,
            pltpu.SemaphoreType.DMA,
        ],
    )(x, pk)


def _mean_body(a, b, c, d, o):
    o[...] = (a[...] + b[...] + c[...] + d[...]) * 0.25


def _mean4(x0, x1, x2, x3):
    xs = [v.reshape(N // 2, 2 * D) for v in (x0, x1, x2, x3)]
    spec = pl.BlockSpec((5000, 2 * D), lambda i: (i, 0))
    out = pl.pallas_call(
        _mean_body,
        grid=(5,),
        in_specs=[spec] * 4,
        out_specs=spec,
        out_shape=jax.ShapeDtypeStruct((N // 2, 2 * D), jnp.float32),
    )(*xs)
    return out.reshape(N, D)


def kernel(user_emb, item_emb, adj_vals, adj_rows, adj_cols):
    x0 = jnp.concatenate([user_emb, item_emb], axis=0)
    pad = E_PAD - E
    zi = jnp.zeros((pad,), jnp.int32)
    ca = jnp.concatenate([adj_cols, zi]).reshape(NBT_ALLOC, B)
    ra = jnp.concatenate([adj_rows, zi]).reshape(NBT_ALLOC, B)
    va = jnp.concatenate(
        [lax.bitcast_convert_type(adj_vals, jnp.int32), zi]).reshape(NBT_ALLOC, B)
    pk = jnp.stack([ca, ra, va], axis=1).reshape(NBT_ALLOC * REC)
    x1 = _sc_layer(x0, pk)
    x2 = _sc_layer(x1, pk)
    x3 = _sc_layer(x2, pk)
    out = _mean4(x0, x1, x2, x3)
    return (out[:N_USERS], out[N_USERS:])


# speedup vs baseline: 2.1824x; 2.1824x over previous
import jax
import jax.numpy as jnp
from jax import lax
from jax.experimental import pallas as pl
from jax.experimental.pallas import tpu as pltpu
from jax.experimental.pallas import tpu_sc as plsc

N_USERS = 10000
N_ITEMS = 40000
N = N_USERS + N_ITEMS
E = 800000
D = 64

NC = 2
NS = 16
DH = D // NC
ACC_ROWS = N + 48
B = 128
NB = 393
NBT = NB * NS
NBT_ALLOC = NBT + 3 * NS
E_PAD = NBT_ALLOC * B
REC = 3 * B

Z_PER_TILE = ACC_ROWS // NS
CP_PER_TILE = 3120


def _bcast_lane(v16, e):
    idx = jnp.full((16, 1), e, jnp.int32)
    return lax.gather(
        v16, idx,
        dimension_numbers=lax.GatherDimensionNumbers(
            offset_dims=(), collapsed_slice_dims=(0,), start_index_map=(0,)),
        slice_sizes=(1,),
        mode=lax.GatherScatterMode.PROMISE_IN_BOUNDS)


def _layer_body(x, pk, y,
                pkb0, pkb1, pkb2, rowsb0, rowsb1, rowsb2,
                lidxb0, lidxb1, lidxb2, zbuf, acc,
                stg0, stg1, stg2, gat0, gat1, gat2, sct0, sct1, sct2):
    c = lax.axis_index("c")
    s = lax.axis_index("s")
    xbase = c * N

    pkb = (pkb0, pkb1, pkb2)
    rowsb = (rowsb0, rowsb1, rowsb2)
    lidxb = (lidxb0, lidxb1, lidxb2)
    stg = (stg0, stg1, stg2)
    gat = (gat0, gat1, gat2)
    sct = (sct0, sct1, sct2)

    def zrow(r, _):
        for k in range(2):
            zbuf[r, pl.ds(k * 16, 16)] = jnp.zeros((16,), jnp.float32)
        return 0
    lax.fori_loop(0, 64, zrow, 0)
    z0 = s * Z_PER_TILE
    def zcopy(i, _):
        pltpu.sync_copy(zbuf, acc.at[pl.ds(z0 + i * 64, 64)])
        return 0
    lax.fori_loop(0, 48, zcopy, 0)
    pltpu.sync_copy(zbuf.at[pl.ds(0, 56)], acc.at[pl.ds(z0 + 3072, 56)])
    plsc.subcore_barrier()

    def stage(j, q):
        pltpu.async_copy(pk.at[pl.ds((s + j * NS) * REC, REC)],
                         pkb[q], stg[q])

    def wait_stage(q):
        pltpu.make_async_copy(pk.at[pl.ds(0, REC)], pkb[q], stg[q]).wait()

    def adjust(q):
        for g in range(B // 16):
            gb = g * 16
            pkb[q][pl.ds(gb, 16)] = pkb[q][pl.ds(gb, 16)] + xbase

    def gather(q):
        pltpu.async_copy(x.at[pkb[q].at[pl.ds(0, B)]], rowsb[q], gat[q])

    def wait_gather(q):
        pltpu.make_async_copy(x.at[pkb[q].at[pl.ds(0, B)]], rowsb[q],
                              gat[q]).wait()

    def scatter(q):
        pltpu.async_copy(rowsb[q], acc.at[lidxb[q]], sct[q], add=True)

    def wait_scatter(q):
        pltpu.make_async_copy(rowsb[q], acc.at[lidxb[q]], sct[q]).wait()

    def compute(q):
        for g in range(B // 16):
            gb = g * 16
            lidxb[q][pl.ds(gb, 16)] = pkb[q][pl.ds(B + gb, 16)]
            v16 = lax.bitcast_convert_type(
                pkb[q][pl.ds(2 * B + gb, 16)], jnp.float32)
            for e in range(16):
                sv = _bcast_lane(v16, e)
                r = gb + e
                for k in range(2):
                    rowsb[q][r, pl.ds(k * 16, 16)] = (
                        rowsb[q][r, pl.ds(k * 16, 16)] * sv)

    stage(0, 0)
    stage(1, 1)
    stage(2, 2)
    wait_stage(0)
    adjust(0)
    gather(0)
    wait_stage(1)
    adjust(1)
    gather(1)

    def triple(i, _):
        for k in range(3):
            qn = (k + 2) % 3
            wait_gather(k)
            compute(k)
            scatter(k)
            if k == 0:
                @pl.when(i > 0)
                def _():
                    wait_scatter(qn)
            else:
                wait_scatter(qn)
            wait_stage(qn)
            adjust(qn)
            if k == 0:
                gather(qn)
            else:
                @pl.when(i < NB // 3 - 1)
                def _():
                    gather(qn)
            stage(3 * i + k + 3, k)
        return 0
    lax.fori_loop(0, NB // 3, triple, 0)

    wait_scatter((NB - 1) % 3)
    wait_stage((NB + 2) % 3)

    plsc.subcore_barrier()
    r0 = s * CP_PER_TILE
    pltpu.sync_copy(acc.at[pl.ds(r0, CP_PER_TILE)],
                    y.at[pl.ds(xbase + r0, CP_PER_TILE)])
    @pl.when(s == 0)
    def _():
        pltpu.sync_copy(acc.at[pl.ds(NS * CP_PER_TILE, 80)],
                        y.at[pl.ds(xbase + NS * CP_PER_TILE, 80)])


def _sc_layer(x, pk):
    mesh = plsc.VectorSubcoreMesh(
        core_axis_name="c", subcore_axis_name="s",
        num_cores=NC, num_subcores=NS)
    return pl.kernel(
        _layer_body,
        out_type=jax.ShapeDtypeStruct((NC * N, DH), jnp.float32),
        mesh=mesh,
        compiler_params=pltpu.CompilerParams(use_tc_tiling_on_sc=False),
        scratch_types=[
            pltpu.VMEM((REC,), jnp.int32),
            pltpu.VMEM((REC,), jnp.int32),
            pltpu.VMEM((REC,), jnp.int32),
            pltpu.VMEM((B, DH), jnp.float32),
            pltpu.VMEM((B, DH), jnp.float32),
            pltpu.VMEM((B, DH), jnp.float32),
            pltpu.VMEM((B,), jnp.int32),
            pltpu.VMEM((B,), jnp.int32),
            pltpu.VMEM((B,), jnp.int32),
            pltpu.VMEM((64, DH), jnp.float32),
            pltpu.VMEM_SHARED((ACC_ROWS, DH), jnp.float32),
            pltpu.SemaphoreType.DMA,
            pltpu.SemaphoreType.DMA,
            pltpu.SemaphoreType.DMA,
            pltpu.SemaphoreType.DMA,
            pltpu.SemaphoreType.DMA,
            pltpu.SemaphoreType.DMA,
            pltpu.SemaphoreType.DMA,
            pltpu.SemaphoreType.DMA,
            pltpu.SemaphoreType.DMA,
        ],
    )(x, pk)


def _mean_body(a, b, c, d, o):
    o[...] = (a[...] + b[...] + c[...] + d[...]) * 0.25


def _mean4(x0, x1, x2, x3):
    xs = [v.reshape(25000, 128) for v in (x0, x1, x2, x3)]
    spec = pl.BlockSpec((5000, 128), lambda i: (i, 0))
    out = pl.pallas_call(
        _mean_body,
        grid=(5,),
        in_specs=[spec] * 4,
        out_specs=spec,
        out_shape=jax.ShapeDtypeStruct((25000, 128), jnp.float32),
    )(*xs)
    return out.reshape(NC * N, DH)


def kernel(user_emb, item_emb, adj_vals, adj_rows, adj_cols):
    x0 = jnp.concatenate([user_emb, item_emb], axis=0)
    x0h = jnp.concatenate([x0[:, :DH], x0[:, DH:]], axis=0)
    pad = E_PAD - E
    zi = jnp.zeros((pad,), jnp.int32)
    ca = jnp.concatenate([adj_cols, zi]).reshape(NBT_ALLOC, B)
    ra = jnp.concatenate([adj_rows, zi]).reshape(NBT_ALLOC, B)
    va = jnp.concatenate(
        [lax.bitcast_convert_type(adj_vals, jnp.int32), zi]
    ).reshape(NBT_ALLOC, B)
    pk = jnp.stack([ca, ra, va], axis=1).reshape(NBT_ALLOC * REC)
    x1h = _sc_layer(x0h, pk)
    x2h = _sc_layer(x1h, pk)
    x3h = _sc_layer(x2h, pk)
    outh = _mean4(x0h, x1h, x2h, x3h)
    out = jnp.concatenate([outh[:N], outh[N:]], axis=1)
    return (out[:N_USERS], out[N_USERS:])


# speedup vs baseline: 2.2974x; 1.0527x over previous
import jax
import jax.numpy as jnp
from jax import lax
from jax.experimental import pallas as pl
from jax.experimental.pallas import tpu as pltpu
from jax.experimental.pallas import tpu_sc as plsc

N_USERS = 10000
N_ITEMS = 40000
N = N_USERS + N_ITEMS
E = 800000
D = 64

NC = 2
NS = 16
DH = D // NC
ACC_ROWS = N + 48
B = 128
NB = 392
NBT = NB * NS
NBT_ALLOC = NBT + 4 * NS
E_PAD = NBT_ALLOC * B
REC = 3 * B

Z_PER_TILE = ACC_ROWS // NS
CP_PER_TILE = 3120


def _bcast_lane(v16, e):
    idx = jnp.full((16, 1), e, jnp.int32)
    return lax.gather(
        v16, idx,
        dimension_numbers=lax.GatherDimensionNumbers(
            offset_dims=(), collapsed_slice_dims=(0,), start_index_map=(0,)),
        slice_sizes=(1,),
        mode=lax.GatherScatterMode.PROMISE_IN_BOUNDS)


def _layer_body(x, pk, y,
                pkb0, pkb1, pkb2, pkb3, rowsb0, rowsb1, rowsb2, rowsb3,
                lidxb0, lidxb1, lidxb2, lidxb3, zbuf, acc,
                stg0, stg1, stg2, stg3, gat0, gat1, gat2, gat3,
                sct0, sct1, sct2, sct3):
    c = lax.axis_index("c")
    s = lax.axis_index("s")
    xbase = c * N

    pkb = (pkb0, pkb1, pkb2, pkb3)
    rowsb = (rowsb0, rowsb1, rowsb2, rowsb3)
    lidxb = (lidxb0, lidxb1, lidxb2, lidxb3)
    stg = (stg0, stg1, stg2, stg3)
    gat = (gat0, gat1, gat2, gat3)
    sct = (sct0, sct1, sct2, sct3)

    def zrow(r, _):
        for k in range(2):
            zbuf[r, pl.ds(k * 16, 16)] = jnp.zeros((16,), jnp.float32)
        return 0
    lax.fori_loop(0, 64, zrow, 0)
    z0 = s * Z_PER_TILE
    def zcopy(i, _):
        pltpu.sync_copy(zbuf, acc.at[pl.ds(z0 + i * 64, 64)])
        return 0
    lax.fori_loop(0, 48, zcopy, 0)
    pltpu.sync_copy(zbuf.at[pl.ds(0, 56)], acc.at[pl.ds(z0 + 3072, 56)])
    plsc.subcore_barrier()

    def stage(j, q):
        pltpu.async_copy(pk.at[pl.ds((s + j * NS) * REC, REC)],
                         pkb[q], stg[q])

    def wait_stage(q):
        pltpu.make_async_copy(pk.at[pl.ds(0, REC)], pkb[q], stg[q]).wait()

    def adjust(q):
        for g in range(B // 16):
            gb = g * 16
            pkb[q][pl.ds(gb, 16)] = pkb[q][pl.ds(gb, 16)] + xbase

    def gather(q):
        pltpu.async_copy(x.at[pkb[q].at[pl.ds(0, B)]], rowsb[q], gat[q])

    def wait_gather(q):
        pltpu.make_async_copy(x.at[pkb[q].at[pl.ds(0, B)]], rowsb[q],
                              gat[q]).wait()

    def scatter(q):
        pltpu.async_copy(rowsb[q], acc.at[lidxb[q]], sct[q], add=True)

    def wait_scatter(q):
        pltpu.make_async_copy(rowsb[q], acc.at[lidxb[q]], sct[q]).wait()

    def compute(q):
        for g in range(B // 16):
            gb = g * 16
            lidxb[q][pl.ds(gb, 16)] = pkb[q][pl.ds(B + gb, 16)]
            v16 = lax.bitcast_convert_type(
                pkb[q][pl.ds(2 * B + gb, 16)], jnp.float32)
            for e in range(16):
                sv = _bcast_lane(v16, e)
                r = gb + e
                for k in range(2):
                    rowsb[q][r, pl.ds(k * 16, 16)] = (
                        rowsb[q][r, pl.ds(k * 16, 16)] * sv)

    for q in range(4):
        stage(q, q)
    for q in range(3):
        wait_stage(q)
        adjust(q)
        gather(q)

    def quad(i, _):
        for k in range(4):
            qn = (k + 3) % 4
            wait_gather(k)
            compute(k)
            scatter(k)
            if k == 0:
                @pl.when(i > 0)
                def _():
                    wait_scatter(qn)
            else:
                wait_scatter(qn)
            wait_stage(qn)
            adjust(qn)
            if k == 0:
                gather(qn)
            else:
                @pl.when(i < NB // 4 - 1)
                def _():
                    gather(qn)
            stage(4 * i + k + 4, k)
        return 0
    lax.fori_loop(0, NB // 4, quad, 0)

    wait_scatter((NB - 1) % 4)
    wait_stage((NB + 3) % 4)

    plsc.subcore_barrier()
    r0 = s * CP_PER_TILE
    pltpu.sync_copy(acc.at[pl.ds(r0, CP_PER_TILE)],
                    y.at[pl.ds(xbase + r0, CP_PER_TILE)])
    @pl.when(s == 0)
    def _():
        pltpu.sync_copy(acc.at[pl.ds(NS * CP_PER_TILE, 80)],
                        y.at[pl.ds(xbase + NS * CP_PER_TILE, 80)])


def _sc_layer(x, pk):
    mesh = plsc.VectorSubcoreMesh(
        core_axis_name="c", subcore_axis_name="s",
        num_cores=NC, num_subcores=NS)
    return pl.kernel(
        _layer_body,
        out_type=jax.ShapeDtypeStruct((NC * N, DH), jnp.float32),
        mesh=mesh,
        compiler_params=pltpu.CompilerParams(use_tc_tiling_on_sc=False),
        scratch_types=[
            pltpu.VMEM((REC,), jnp.int32),
            pltpu.VMEM((REC,), jnp.int32),
            pltpu.VMEM((REC,), jnp.int32),
            pltpu.VMEM((REC,), jnp.int32),
            pltpu.VMEM((B, DH), jnp.float32),
            pltpu.VMEM((B, DH), jnp.float32),
            pltpu.VMEM((B, DH), jnp.float32),
            pltpu.VMEM((B, DH), jnp.float32),
            pltpu.VMEM((B,), jnp.int32),
            pltpu.VMEM((B,), jnp.int32),
            pltpu.VMEM((B,), jnp.int32),
            pltpu.VMEM((B,), jnp.int32),
            pltpu.VMEM((64, DH), jnp.float32),
            pltpu.VMEM_SHARED((ACC_ROWS, DH), jnp.float32),
            pltpu.SemaphoreType.DMA,
            pltpu.SemaphoreType.DMA,
            pltpu.SemaphoreType.DMA,
            pltpu.SemaphoreType.DMA,
            pltpu.SemaphoreType.DMA,
            pltpu.SemaphoreType.DMA,
            pltpu.SemaphoreType.DMA,
            pltpu.SemaphoreType.DMA,
            pltpu.SemaphoreType.DMA,
            pltpu.SemaphoreType.DMA,
            pltpu.SemaphoreType.DMA,
            pltpu.SemaphoreType.DMA,
        ],
    )(x, pk)


def _mean_body(a, b, c, d, o):
    o[...] = (a[...] + b[...] + c[...] + d[...]) * 0.25


def _mean4(x0, x1, x2, x3):
    xs = [v.reshape(25000, 128) for v in (x0, x1, x2, x3)]
    spec = pl.BlockSpec((5000, 128), lambda i: (i, 0))
    out = pl.pallas_call(
        _mean_body,
        grid=(5,),
        in_specs=[spec] * 4,
        out_specs=spec,
        out_shape=jax.ShapeDtypeStruct((25000, 128), jnp.float32),
    )(*xs)
    return out.reshape(NC * N, DH)


def kernel(user_emb, item_emb, adj_vals, adj_rows, adj_cols):
    x0 = jnp.concatenate([user_emb, item_emb], axis=0)
    x0h = jnp.concatenate([x0[:, :DH], x0[:, DH:]], axis=0)
    pad = E_PAD - E
    zi = jnp.zeros((pad,), jnp.int32)
    ca = jnp.concatenate([adj_cols, zi]).reshape(NBT_ALLOC, B)
    ra = jnp.concatenate([adj_rows, zi]).reshape(NBT_ALLOC, B)
    va = jnp.concatenate(
        [lax.bitcast_convert_type(adj_vals, jnp.int32), zi]
    ).reshape(NBT_ALLOC, B)
    pk = jnp.stack([ca, ra, va], axis=1).reshape(NBT_ALLOC * REC)
    x1h = _sc_layer(x0h, pk)
    x2h = _sc_layer(x1h, pk)
    x3h = _sc_layer(x2h, pk)
    outh = _mean4(x0h, x1h, x2h, x3h)
    out = jnp.concatenate([outh[:N], outh[N:]], axis=1)
    return (out[:N_USERS], out[N_USERS:])


# speedup vs baseline: 2.3002x; 1.0012x over previous
import jax
import jax.numpy as jnp
from jax import lax
from jax.experimental import pallas as pl
from jax.experimental.pallas import tpu as pltpu
from jax.experimental.pallas import tpu_sc as plsc

N_USERS = 10000
N_ITEMS = 40000
N = N_USERS + N_ITEMS
E = 800000
D = 64

NC = 2
NS = 16
DH = D // NC
ACC_ROWS = N + 48
B = 128
NB = 392
NBT = NB * NS
NBT_ALLOC = NBT + 4 * NS
E_PAD = NBT_ALLOC * B
REC = 3 * B

Z_PER_TILE = ACC_ROWS // NS
CP_PER_TILE = 3120


def _bcast_lane(v16, e):
    idx = jnp.full((16, 1), e, jnp.int32)
    return lax.gather(
        v16, idx,
        dimension_numbers=lax.GatherDimensionNumbers(
            offset_dims=(), collapsed_slice_dims=(0,), start_index_map=(0,)),
        slice_sizes=(1,),
        mode=lax.GatherScatterMode.PROMISE_IN_BOUNDS)


def _layer_body(x, pk, y,
                pkb0, pkb1, pkb2, pkb3, rowsb0, rowsb1, rowsb2, rowsb3,
                lidxb0, lidxb1, lidxb2, lidxb3, zbuf, acc,
                stg0, stg1, stg2, stg3, gat0, gat1, gat2, gat3,
                sct0, sct1, sct2, sct3):
    c = lax.axis_index("c")
    s = lax.axis_index("s")
    xbase = c * N

    pkb = (pkb0, pkb1, pkb2, pkb3)
    rowsb = (rowsb0, rowsb1, rowsb2, rowsb3)
    lidxb = (lidxb0, lidxb1, lidxb2, lidxb3)
    stg = (stg0, stg1, stg2, stg3)
    gat = (gat0, gat1, gat2, gat3)
    sct = (sct0, sct1, sct2, sct3)

    def zrow(r, _):
        for k in range(2):
            zbuf[r, pl.ds(k * 16, 16)] = jnp.zeros((16,), jnp.float32)
        return 0
    lax.fori_loop(0, 128, zrow, 0)
    z0 = s * Z_PER_TILE
    def zcopy(i, _):
        pltpu.sync_copy(zbuf, acc.at[pl.ds(z0 + i * 128, 128)])
        return 0
    lax.fori_loop(0, 24, zcopy, 0)
    pltpu.sync_copy(zbuf.at[pl.ds(0, 56)], acc.at[pl.ds(z0 + 3072, 56)])
    plsc.subcore_barrier()

    def stage(j, q):
        pltpu.async_copy(pk.at[pl.ds((s + j * NS) * REC, REC)],
                         pkb[q], stg[q])

    def wait_stage(q):
        pltpu.make_async_copy(pk.at[pl.ds(0, REC)], pkb[q], stg[q]).wait()

    def adjust(q):
        for g in range(B // 16):
            gb = g * 16
            pkb[q][pl.ds(gb, 16)] = pkb[q][pl.ds(gb, 16)] + xbase

    def gather(q):
        pltpu.async_copy(x.at[pkb[q].at[pl.ds(0, B)]], rowsb[q], gat[q])

    def wait_gather(q):
        pltpu.make_async_copy(x.at[pkb[q].at[pl.ds(0, B)]], rowsb[q],
                              gat[q]).wait()

    def scatter(q):
        pltpu.async_copy(rowsb[q], acc.at[lidxb[q]], sct[q], add=True)

    def wait_scatter(q):
        pltpu.make_async_copy(rowsb[q], acc.at[lidxb[q]], sct[q]).wait()

    def compute(q):
        for g in range(B // 16):
            gb = g * 16
            lidxb[q][pl.ds(gb, 16)] = pkb[q][pl.ds(B + gb, 16)]
            v16 = lax.bitcast_convert_type(
                pkb[q][pl.ds(2 * B + gb, 16)], jnp.float32)
            for e in range(16):
                sv = _bcast_lane(v16, e)
                r = gb + e
                for k in range(2):
                    rowsb[q][r, pl.ds(k * 16, 16)] = (
                        rowsb[q][r, pl.ds(k * 16, 16)] * sv)

    for q in range(4):
        stage(q, q)
    for q in range(3):
        wait_stage(q)
        adjust(q)
        gather(q)

    def quad(i, _):
        for k in range(4):
            qn = (k + 3) % 4
            wait_gather(k)
            compute(k)
            scatter(k)
            if k == 0:
                @pl.when(i > 0)
                def _():
                    wait_scatter(qn)
            else:
                wait_scatter(qn)
            wait_stage(qn)
            adjust(qn)
            if k == 0:
                gather(qn)
            else:
                @pl.when(i < NB // 4 - 1)
                def _():
                    gather(qn)
            stage(4 * i + k + 4, k)
        return 0
    lax.fori_loop(0, NB // 4, quad, 0)

    wait_scatter((NB - 1) % 4)
    wait_stage((NB + 3) % 4)

    plsc.subcore_barrier()
    r0 = s * CP_PER_TILE
    pltpu.sync_copy(acc.at[pl.ds(r0, CP_PER_TILE)],
                    y.at[pl.ds(xbase + r0, CP_PER_TILE)])
    @pl.when(s == 0)
    def _():
        pltpu.sync_copy(acc.at[pl.ds(NS * CP_PER_TILE, 80)],
                        y.at[pl.ds(xbase + NS * CP_PER_TILE, 80)])


def _sc_layer(x, pk):
    mesh = plsc.VectorSubcoreMesh(
        core_axis_name="c", subcore_axis_name="s",
        num_cores=NC, num_subcores=NS)
    return pl.kernel(
        _layer_body,
        out_type=jax.ShapeDtypeStruct((NC * N, DH), jnp.float32),
        mesh=mesh,
        compiler_params=pltpu.CompilerParams(use_tc_tiling_on_sc=False),
        scratch_types=[
            pltpu.VMEM((REC,), jnp.int32),
            pltpu.VMEM((REC,), jnp.int32),
            pltpu.VMEM((REC,), jnp.int32),
            pltpu.VMEM((REC,), jnp.int32),
            pltpu.VMEM((B, DH), jnp.float32),
            pltpu.VMEM((B, DH), jnp.float32),
            pltpu.VMEM((B, DH), jnp.float32),
            pltpu.VMEM((B, DH), jnp.float32),
            pltpu.VMEM((B,), jnp.int32),
            pltpu.VMEM((B,), jnp.int32),
            pltpu.VMEM((B,), jnp.int32),
            pltpu.VMEM((B,), jnp.int32),
            pltpu.VMEM((128, DH), jnp.float32),
            pltpu.VMEM_SHARED((ACC_ROWS, DH), jnp.float32),
            pltpu.SemaphoreType.DMA,
            pltpu.SemaphoreType.DMA,
            pltpu.SemaphoreType.DMA,
            pltpu.SemaphoreType.DMA,
            pltpu.SemaphoreType.DMA,
            pltpu.SemaphoreType.DMA,
            pltpu.SemaphoreType.DMA,
            pltpu.SemaphoreType.DMA,
            pltpu.SemaphoreType.DMA,
            pltpu.SemaphoreType.DMA,
            pltpu.SemaphoreType.DMA,
            pltpu.SemaphoreType.DMA,
        ],
    )(x, pk)


def _mean_body(a, b, c, d, o):
    o[...] = (a[...] + b[...] + c[...] + d[...]) * 0.25


def _mean4(x0, x1, x2, x3):
    xs = [v.reshape(25000, 128) for v in (x0, x1, x2, x3)]
    spec = pl.BlockSpec((5000, 128), lambda i: (i, 0))
    out = pl.pallas_call(
        _mean_body,
        grid=(5,),
        in_specs=[spec] * 4,
        out_specs=spec,
        out_shape=jax.ShapeDtypeStruct((25000, 128), jnp.float32),
    )(*xs)
    return out.reshape(NC * N, DH)


def kernel(user_emb, item_emb, adj_vals, adj_rows, adj_cols):
    x0 = jnp.concatenate([user_emb, item_emb], axis=0)
    x0h = jnp.concatenate([x0[:, :DH], x0[:, DH:]], axis=0)
    pad = E_PAD - E
    zi = jnp.zeros((pad,), jnp.int32)
    ca = jnp.concatenate([adj_cols, zi]).reshape(NBT_ALLOC, B)
    ra = jnp.concatenate([adj_rows, zi]).reshape(NBT_ALLOC, B)
    va = jnp.concatenate(
        [lax.bitcast_convert_type(adj_vals, jnp.int32), zi]
    ).reshape(NBT_ALLOC, B)
    pk = jnp.stack([ca, ra, va], axis=1).reshape(NBT_ALLOC * REC)
    x1h = _sc_layer(x0h, pk)
    x2h = _sc_layer(x1h, pk)
    x3h = _sc_layer(x2h, pk)
    outh = _mean4(x0h, x1h, x2h, x3h)
    out = jnp.concatenate([outh[:N], outh[N:]], axis=1)
    return (out[:N_USERS], out[N_USERS:])
